# baseline (speedup 1.0000x reference)
import functools

import jax
import jax.numpy as jnp
from jax import lax
from jax.experimental import pallas as pl
from jax.experimental.pallas import tpu as pltpu
from jax.experimental.pallas import tpu_sc as plsc

_NUM_CORES = 2
_NUM_SUBCORES = 16
_NUM_WORKERS = _NUM_CORES * _NUM_SUBCORES


@functools.partial(jax.jit, static_argnames=("rows_per_w", "chunk"))
def _sc_gather(W, x, *, rows_per_w, chunk):
    n_chunks = rows_per_w // chunk
    R, S = x.shape
    D = W.shape[1]
    mesh = plsc.VectorSubcoreMesh(core_axis_name="c", subcore_axis_name="s")

    @functools.partial(
        pl.kernel,
        mesh=mesh,
        out_type=jax.ShapeDtypeStruct((R, S, D), jnp.float32),
        scratch_types=[
            pltpu.VMEM((chunk, S), jnp.int32),
            pltpu.VMEM((chunk, S, D), jnp.float32),
            pltpu.SemaphoreType.DMA,
        ],
        compiler_params=pltpu.CompilerParams(use_tc_tiling_on_sc=False),
    )
    def k(table_hbm, x_hbm, out_hbm, idx_v, rows_v, sem):
        wid = lax.axis_index("s") * _NUM_CORES + lax.axis_index("c")
        wbase = wid * rows_per_w

        def body(i, _):
            rbase = wbase + i * chunk
            pltpu.sync_copy(x_hbm.at[pl.ds(rbase, chunk)], idx_v)
            copies = [
                pltpu.async_copy(table_hbm.at[idx_v.at[j]], rows_v.at[j], sem)
                for j in range(chunk)
            ]
            for c in copies:
                c.wait()
            pltpu.sync_copy(rows_v, out_hbm.at[pl.ds(rbase, chunk)])
            return 0

        lax.fori_loop(0, n_chunks, body, 0)

    return k(W, x)


def kernel(W, x):
    R, S = x.shape
    xi = x.astype(jnp.int32)
    assert R % _NUM_WORKERS == 0
    rows_per_w = R // _NUM_WORKERS
    chunk = 8
    assert rows_per_w % chunk == 0
    return _sc_gather(W, xi, rows_per_w=rows_per_w, chunk=chunk)


# speedup vs baseline: 1.0518x; 1.0518x over previous
import functools

import jax
import jax.numpy as jnp
from jax import lax
from jax.experimental import pallas as pl
from jax.experimental.pallas import tpu as pltpu
from jax.experimental.pallas import tpu_sc as plsc

_NUM_CORES = 2
_NUM_SUBCORES = 16
_NUM_WORKERS = _NUM_CORES * _NUM_SUBCORES


@functools.partial(jax.jit, static_argnames=("rows_per_w", "chunk"))
def _sc_gather(W, x, *, rows_per_w, chunk):
    n_pairs = rows_per_w // (2 * chunk)
    R, S = x.shape
    D = W.shape[1]
    mesh = plsc.VectorSubcoreMesh(core_axis_name="c", subcore_axis_name="s")

    @functools.partial(
        pl.kernel,
        mesh=mesh,
        out_type=jax.ShapeDtypeStruct((R, S, D), jnp.float32),
        scratch_types=[
            pltpu.VMEM((rows_per_w, S), jnp.int32),
            pltpu.VMEM((2, chunk, S, D), jnp.float32),
            pltpu.SemaphoreType.DMA,
            pltpu.SemaphoreType.DMA,
            pltpu.SemaphoreType.DMA,
        ],
        compiler_params=pltpu.CompilerParams(use_tc_tiling_on_sc=False),
    )
    def k(table_hbm, x_hbm, out_hbm, idx_v, rows_v, gsem, wsem0, wsem1):
        wid = lax.axis_index("s") * _NUM_CORES + lax.axis_index("c")
        wbase = wid * rows_per_w
        pltpu.sync_copy(x_hbm.at[pl.ds(wbase, rows_per_w)], idx_v)
        wsems = (wsem0, wsem1)

        def body(g, _):
            for p in range(2):
                c = 2 * g + p
                buf = rows_v.at[p]

                @pl.when(g > 0)
                def _():
                    pltpu.make_async_copy(
                        out_hbm.at[pl.ds(0, chunk)], buf, wsems[p]
                    ).wait()

                copies = [
                    pltpu.async_copy(
                        table_hbm.at[idx_v.at[c * chunk + j]], buf.at[j], gsem
                    )
                    for j in range(chunk)
                ]
                for cp in copies:
                    cp.wait()
                pltpu.async_copy(
                    buf, out_hbm.at[pl.ds(wbase + c * chunk, chunk)], wsems[p]
                )
            return 0

        lax.fori_loop(0, n_pairs, body, 0)
        for p in range(2):
            pltpu.make_async_copy(
                out_hbm.at[pl.ds(0, chunk)], rows_v.at[p], wsems[p]
            ).wait()

    return k(W, x)


def kernel(W, x):
    R, S = x.shape
    xi = x.astype(jnp.int32)
    assert R % _NUM_WORKERS == 0
    rows_per_w = R // _NUM_WORKERS
    chunk = 8
    assert rows_per_w % (2 * chunk) == 0
    return _sc_gather(W, xi, rows_per_w=rows_per_w, chunk=chunk)


# speedup vs baseline: 1.0705x; 1.0178x over previous
import functools

import jax
import jax.numpy as jnp
from jax import lax
from jax.experimental import pallas as pl
from jax.experimental.pallas import tpu as pltpu
from jax.experimental.pallas import tpu_sc as plsc

_NUM_CORES = 2
_NUM_SUBCORES = 16
_NUM_WORKERS = _NUM_CORES * _NUM_SUBCORES


@functools.partial(jax.jit, static_argnames=("rows_per_w", "chunk"))
def _sc_gather(W, x, *, rows_per_w, chunk):
    n_pairs = rows_per_w // (2 * chunk)
    R, S = x.shape
    D = W.shape[1]
    mesh = plsc.VectorSubcoreMesh(core_axis_name="c", subcore_axis_name="s")

    @functools.partial(
        pl.kernel,
        mesh=mesh,
        out_type=jax.ShapeDtypeStruct((R, S, D), jnp.float32),
        scratch_types=[
            pltpu.VMEM((rows_per_w, S), jnp.int32),
            pltpu.VMEM((2, chunk, S, D), jnp.float32),
            pltpu.SemaphoreType.DMA,
            pltpu.SemaphoreType.DMA,
            pltpu.SemaphoreType.DMA,
        ],
        compiler_params=pltpu.CompilerParams(use_tc_tiling_on_sc=False),
    )
    def k(table_hbm, x_hbm, out_hbm, idx_v, rows_v, gsem, wsem0, wsem1):
        wid = lax.axis_index("s") * _NUM_CORES + lax.axis_index("c")
        wbase = wid * rows_per_w
        pltpu.sync_copy(x_hbm.at[pl.ds(wbase, rows_per_w)], idx_v)
        wsems = (wsem0, wsem1)

        def body(g, _):
            for p in range(2):
                c = 2 * g + p
                buf = rows_v.at[p]

                @pl.when(g > 0)
                def _():
                    pltpu.make_async_copy(
                        out_hbm.at[pl.ds(0, chunk)], buf, wsems[p]
                    ).wait()

                copies = [
                    pltpu.async_copy(
                        table_hbm.at[idx_v.at[c * chunk + j]], buf.at[j], gsem
                    )
                    for j in range(chunk)
                ]
                for cp in copies:
                    cp.wait()
                pltpu.async_copy(
                    buf, out_hbm.at[pl.ds(wbase + c * chunk, chunk)], wsems[p]
                )
            return 0

        lax.fori_loop(0, n_pairs, body, 0)
        for p in range(2):
            pltpu.make_async_copy(
                out_hbm.at[pl.ds(0, chunk)], rows_v.at[p], wsems[p]
            ).wait()

    return k(W, x)


def kernel(W, x):
    R, S = x.shape
    xi = x.astype(jnp.int32)
    assert R % _NUM_WORKERS == 0
    rows_per_w = R // _NUM_WORKERS
    chunk = 16
    assert rows_per_w % (2 * chunk) == 0
    return _sc_gather(W, xi, rows_per_w=rows_per_w, chunk=chunk)


# speedup vs baseline: 1.0723x; 1.0017x over previous
import functools

import jax
import jax.numpy as jnp
from jax import lax
from jax.experimental import pallas as pl
from jax.experimental.pallas import tpu as pltpu
from jax.experimental.pallas import tpu_sc as plsc

_NUM_CORES = 2
_NUM_SUBCORES = 16
_NUM_WORKERS = _NUM_CORES * _NUM_SUBCORES


@functools.partial(jax.jit, static_argnames=("rows_per_w", "chunk"))
def _sc_gather(W, x, *, rows_per_w, chunk):
    n_pairs = rows_per_w // (2 * chunk)
    R, S = x.shape
    D = W.shape[1]
    mesh = plsc.VectorSubcoreMesh(core_axis_name="c", subcore_axis_name="s")

    @functools.partial(
        pl.kernel,
        mesh=mesh,
        out_type=jax.ShapeDtypeStruct((R, S, D), jnp.float32),
        scratch_types=[
            pltpu.VMEM((rows_per_w, S), jnp.int32),
            pltpu.VMEM((2, chunk, S, D), jnp.float32),
            pltpu.SemaphoreType.DMA,
            pltpu.SemaphoreType.DMA,
            pltpu.SemaphoreType.DMA,
            pltpu.SemaphoreType.DMA,
        ],
        compiler_params=pltpu.CompilerParams(use_tc_tiling_on_sc=False),
    )
    def k(table_hbm, x_hbm, out_hbm, idx_v, rows_v, gsem0, gsem1, wsem0, wsem1):
        wid = lax.axis_index("s") * _NUM_CORES + lax.axis_index("c")
        wbase = wid * rows_per_w
        pltpu.sync_copy(x_hbm.at[pl.ds(wbase, rows_per_w)], idx_v)
        wsems = (wsem0, wsem1)
        gsems = (gsem0, gsem1)

        def body(g, _):
            for p in range(2):
                c = 2 * g + p
                buf = rows_v.at[p]

                @pl.when(g > 0)
                def _():
                    pltpu.make_async_copy(
                        out_hbm.at[pl.ds(0, chunk)], buf, wsems[p]
                    ).wait()

                for j in range(chunk):
                    pltpu.async_copy(
                        table_hbm.at[idx_v.at[c * chunk + j]], buf.at[j], gsems[p]
                    )
            for p in range(2):
                c = 2 * g + p
                buf = rows_v.at[p]
                pltpu.make_async_copy(
                    out_hbm.at[pl.ds(0, chunk)], buf, gsems[p]
                ).wait()
                pltpu.async_copy(
                    buf, out_hbm.at[pl.ds(wbase + c * chunk, chunk)], wsems[p]
                )
            return 0

        lax.fori_loop(0, n_pairs, body, 0)
        for p in range(2):
            pltpu.make_async_copy(
                out_hbm.at[pl.ds(0, chunk)], rows_v.at[p], wsems[p]
            ).wait()

    return k(W, x)


def kernel(W, x):
    R, S = x.shape
    xi = x.astype(jnp.int32)
    assert R % _NUM_WORKERS == 0
    rows_per_w = R // _NUM_WORKERS
    chunk = 16
    assert rows_per_w % (2 * chunk) == 0
    return _sc_gather(W, xi, rows_per_w=rows_per_w, chunk=chunk)
